# baseline (speedup 1.0000x reference)
import functools

import jax
import jax.numpy as jnp
from jax import lax
from jax.experimental import pallas as pl
from jax.experimental.pallas import tpu as pltpu
from jax.experimental.pallas import tpu_sc as plsc

_B, _N, _D = 8, 4096, 768
_C = 16
_NUM_CLASSES = 2
_THR = 0.8

_NSC = 1024
_NTC = _N - _NSC
_NCORE, _NSUB = 2, 16
_WPB = (_NCORE * _NSUB) // _B
_RPW = _NSC // _WPB
_RBLK = 64
_NBLK = _RPW // _RBLK
_LANES = 16
_DCH = _D // _LANES


def _sc_body(x_hbm, c_hbm, out_hbm, xb0, xb1, ib0, ib1, acc, sem0, sem1):
    core = lax.axis_index("c")
    sid = lax.axis_index("s")
    batch = sid % _B
    chunk = (sid // _B) + _NCORE * core
    row0 = _NTC + chunk * _RPW

    @pl.loop(0, _C)
    def _zr(r):
        for j in range(_DCH):
            acc[r, pl.ds(j * _LANES, _LANES)] = jnp.zeros(
                (_LANES,), jnp.float32
            )

    def start(i, xb, ib, sem):
        base = row0 + i * _RBLK
        dx = pltpu.async_copy(x_hbm.at[batch, pl.ds(base, _RBLK)], xb, sem)
        di = pltpu.async_copy(c_hbm.at[batch, pl.ds(base, _RBLK)], ib, sem)
        return dx, di

    def accumulate(xb, ib):
        @pl.loop(0, _RBLK, step=_LANES)
        def _rows(r16):
            idvec = ib[pl.ds(r16, _LANES)]
            for k in range(_LANES):
                c = idvec[k]
                for j in range(_DCH):
                    plsc.addupdate(
                        acc.at[c, pl.ds(j * _LANES, _LANES)],
                        xb[r16 + k, pl.ds(j * _LANES, _LANES)],
                    )

    def wait(xb, ib, sem):
        pltpu.make_async_copy(
            x_hbm.at[batch, pl.ds(row0, _RBLK)], xb, sem
        ).wait()
        pltpu.make_async_copy(
            c_hbm.at[batch, pl.ds(row0, _RBLK)], ib, sem
        ).wait()

    start(0, xb0, ib0, sem0)

    @pl.loop(0, _NBLK, step=2)
    def _blk(i):
        @pl.when(i + 1 < _NBLK)
        def _s1():
            start(i + 1, xb1, ib1, sem1)

        wait(xb0, ib0, sem0)
        accumulate(xb0, ib0)

        @pl.when(i + 2 < _NBLK)
        def _s2():
            start(i + 2, xb0, ib0, sem0)

        wait(xb1, ib1, sem1)
        accumulate(xb1, ib1)

    pltpu.sync_copy(acc, out_hbm.at[chunk, batch])


def _sc_run(inst_feat, clusters):
    return pl.kernel(
        _sc_body,
        mesh=plsc.VectorSubcoreMesh(core_axis_name="c", subcore_axis_name="s"),
        out_type=jax.ShapeDtypeStruct((_WPB, _B, _C, _D), jnp.float32),
        scratch_types=[
            pltpu.VMEM((_RBLK, _D), jnp.float32),
            pltpu.VMEM((_RBLK, _D), jnp.float32),
            pltpu.VMEM((_RBLK,), jnp.int32),
            pltpu.VMEM((_RBLK,), jnp.int32),
            pltpu.VMEM((_C, _D), jnp.float32),
            pltpu.SemaphoreType.DMA,
            pltpu.SemaphoreType.DMA,
        ],
    )(inst_feat, clusters)


def _tc_body(c_ref, x_ref, o_ref):
    x = x_ref[0]
    cid = jax.lax.broadcasted_iota(jnp.int32, (_C, _NTC), 0)
    mask = (c_ref[0] == cid).astype(jnp.bfloat16)
    x_hi = x.astype(jnp.bfloat16)
    x_lo = (x - x_hi.astype(jnp.float32)).astype(jnp.bfloat16)
    o_ref[0] = jnp.dot(
        mask, x_hi, preferred_element_type=jnp.float32
    ) + jnp.dot(mask, x_lo, preferred_element_type=jnp.float32)


def _tc_run(inst_feat, clusters_tc):
    return pl.pallas_call(
        _tc_body,
        grid=(_B,),
        in_specs=[
            pl.BlockSpec((1, 1, _NTC), lambda i: (i, 0, 0)),
            pl.BlockSpec((1, _NTC, _D), lambda i: (i, 0, 0)),
        ],
        out_specs=pl.BlockSpec((1, _C, _D), lambda i: (i, 0, 0)),
        out_shape=jax.ShapeDtypeStruct((_B, _C, _D), jnp.float32),
    )(clusters_tc, inst_feat)


def _combine_body(c_ref, tsum_ref, ssum_ref, w_ref, b_ref, o_ref):
    w = w_ref[...]
    bias = b_ref[0]
    for bb in range(_B):
        ids = c_ref[pl.ds(bb, 1), :]
        cid = jax.lax.broadcasted_iota(jnp.int32, (_C, _N), 0)
        maskf = (ids == cid).astype(jnp.float32)
        counts = jnp.sum(maskf, axis=1, keepdims=True)
        sums = tsum_ref[bb]
        for ch in range(_WPB):
            sums = sums + ssum_ref[ch, bb]
        feats = sums / jnp.maximum(counts, 1.0)
        logits = jnp.dot(feats, w, preferred_element_type=jnp.float32) + bias
        d = logits[:, 1:2] - logits[:, 0:1]
        dmax = jnp.max(d)
        dmin = jnp.min(d)
        use_min = jax.nn.sigmoid(dmax) < _THR
        target = jnp.where(use_min, dmin, dmax)
        idxs = jax.lax.broadcasted_iota(jnp.int32, (_C, 1), 0)
        sel = jnp.min(jnp.where(d == target, idxs, _C))
        selmask = (idxs == sel).astype(jnp.float32)
        o_ref[pl.ds(bb, 1), :] = jnp.sum(selmask * logits, axis=0,
                                         keepdims=True)


def _combine(clusters, tc_sums, sc_part, W, b2):
    return pl.pallas_call(
        _combine_body,
        grid=(1,),
        in_specs=[
            pl.BlockSpec((_B, _N), lambda i: (0, 0)),
            pl.BlockSpec((_B, _C, _D), lambda i: (0, 0, 0)),
            pl.BlockSpec((_WPB, _B, _C, _D), lambda i: (0, 0, 0, 0)),
            pl.BlockSpec((_D, _NUM_CLASSES), lambda i: (0, 0)),
            pl.BlockSpec((1, _NUM_CLASSES), lambda i: (0, 0)),
        ],
        out_specs=pl.BlockSpec((_B, _NUM_CLASSES), lambda i: (0, 0)),
        out_shape=jax.ShapeDtypeStruct((_B, _NUM_CLASSES), jnp.float32),
    )(clusters, tc_sums, sc_part, W, b2)


@jax.jit
def _run_all(inst_feat, clusters_idcs, W, b):
    cl = clusters_idcs.astype(jnp.int32)
    sc_part = _sc_run(inst_feat, cl)
    tc_sums = _tc_run(inst_feat, cl[:, :_NTC].reshape(_B, 1, _NTC))
    out = _combine(cl, tc_sums, sc_part, W,
                   b.reshape(1, _NUM_CLASSES).astype(jnp.float32))
    return out


def kernel(inst_feat, clusters_idcs, W, b):
    return _run_all(inst_feat, clusters_idcs, W, b)


# speedup vs baseline: 1.0376x; 1.0376x over previous
import functools

import jax
import jax.numpy as jnp
from jax import lax
from jax.experimental import pallas as pl
from jax.experimental.pallas import tpu as pltpu
from jax.experimental.pallas import tpu_sc as plsc

_B, _N, _D = 8, 4096, 768
_C = 16
_NUM_CLASSES = 2
_THR = 0.8

_NSC = 1024
_NTC = _N - _NSC
_NCORE, _NSUB = 2, 16
_WPB = (_NCORE * _NSUB) // _B
_RPW = _NSC // _WPB
_RBLK = 64
_NBLK = _RPW // _RBLK
_LANES = 16
_DCH = _D // _LANES


def _sc_body(x_hbm, c_hbm, out_hbm, xb0, xb1, ib0, ib1, acc, sem0, sem1):
    core = lax.axis_index("c")
    sid = lax.axis_index("s")
    batch = sid % _B
    chunk = (sid // _B) + _NCORE * core
    row0 = _NTC + chunk * _RPW

    @pl.loop(0, _C)
    def _zr(r):
        for j in range(_DCH):
            acc[r, pl.ds(j * _LANES, _LANES)] = jnp.zeros(
                (_LANES,), jnp.float32
            )

    def start(i, xb, ib, sem):
        base = row0 + i * _RBLK
        dx = pltpu.async_copy(x_hbm.at[batch, pl.ds(base, _RBLK)], xb, sem)
        di = pltpu.async_copy(c_hbm.at[batch, pl.ds(base, _RBLK)], ib, sem)
        return dx, di

    def accumulate(xb, ib):
        @plsc.parallel_loop(0, _RBLK, step=8, unroll=2)
        def _rows(r8):
            idvec = ib[pl.ds(r8, _LANES)]
            for k in range(8):
                c = idvec[k]
                for j in range(_DCH):
                    plsc.addupdate(
                        acc.at[c, pl.ds(j * _LANES, _LANES)],
                        xb[r8 + k, pl.ds(j * _LANES, _LANES)],
                    )

    def wait(xb, ib, sem):
        pltpu.make_async_copy(
            x_hbm.at[batch, pl.ds(row0, _RBLK)], xb, sem
        ).wait()
        pltpu.make_async_copy(
            c_hbm.at[batch, pl.ds(row0, _RBLK)], ib, sem
        ).wait()

    start(0, xb0, ib0, sem0)

    @pl.loop(0, _NBLK, step=2)
    def _blk(i):
        @pl.when(i + 1 < _NBLK)
        def _s1():
            start(i + 1, xb1, ib1, sem1)

        wait(xb0, ib0, sem0)
        accumulate(xb0, ib0)

        @pl.when(i + 2 < _NBLK)
        def _s2():
            start(i + 2, xb0, ib0, sem0)

        wait(xb1, ib1, sem1)
        accumulate(xb1, ib1)

    pltpu.sync_copy(acc, out_hbm.at[chunk, batch])


def _sc_run(inst_feat, clusters):
    return pl.kernel(
        _sc_body,
        mesh=plsc.VectorSubcoreMesh(core_axis_name="c", subcore_axis_name="s"),
        out_type=jax.ShapeDtypeStruct((_WPB, _B, _C, _D), jnp.float32),
        scratch_types=[
            pltpu.VMEM((_RBLK, _D), jnp.float32),
            pltpu.VMEM((_RBLK, _D), jnp.float32),
            pltpu.VMEM((_RBLK,), jnp.int32),
            pltpu.VMEM((_RBLK,), jnp.int32),
            pltpu.VMEM((_C, _D), jnp.float32),
            pltpu.SemaphoreType.DMA,
            pltpu.SemaphoreType.DMA,
        ],
    )(inst_feat, clusters)


def _tc_body(c_ref, x_ref, o_ref):
    x = x_ref[0]
    cid = jax.lax.broadcasted_iota(jnp.int32, (_C, _NTC), 0)
    mask = (c_ref[0] == cid).astype(jnp.bfloat16)
    x_hi = x.astype(jnp.bfloat16)
    x_lo = (x - x_hi.astype(jnp.float32)).astype(jnp.bfloat16)
    o_ref[0] = jnp.dot(
        mask, x_hi, preferred_element_type=jnp.float32
    ) + jnp.dot(mask, x_lo, preferred_element_type=jnp.float32)


def _tc_run(inst_feat, clusters_tc):
    return pl.pallas_call(
        _tc_body,
        grid=(_B,),
        in_specs=[
            pl.BlockSpec((1, 1, _NTC), lambda i: (i, 0, 0)),
            pl.BlockSpec((1, _NTC, _D), lambda i: (i, 0, 0)),
        ],
        out_specs=pl.BlockSpec((1, _C, _D), lambda i: (i, 0, 0)),
        out_shape=jax.ShapeDtypeStruct((_B, _C, _D), jnp.float32),
    )(clusters_tc, inst_feat)


def _combine_body(c_ref, tsum_ref, ssum_ref, w_ref, b_ref, o_ref):
    w = w_ref[...]
    bias = b_ref[0]
    for bb in range(_B):
        ids = c_ref[pl.ds(bb, 1), :]
        cid = jax.lax.broadcasted_iota(jnp.int32, (_C, _N), 0)
        maskf = (ids == cid).astype(jnp.float32)
        counts = jnp.sum(maskf, axis=1, keepdims=True)
        sums = tsum_ref[bb]
        for ch in range(_WPB):
            sums = sums + ssum_ref[ch, bb]
        feats = sums / jnp.maximum(counts, 1.0)
        logits = jnp.dot(feats, w, preferred_element_type=jnp.float32) + bias
        d = logits[:, 1:2] - logits[:, 0:1]
        dmax = jnp.max(d)
        dmin = jnp.min(d)
        use_min = jax.nn.sigmoid(dmax) < _THR
        target = jnp.where(use_min, dmin, dmax)
        idxs = jax.lax.broadcasted_iota(jnp.int32, (_C, 1), 0)
        sel = jnp.min(jnp.where(d == target, idxs, _C))
        selmask = (idxs == sel).astype(jnp.float32)
        o_ref[pl.ds(bb, 1), :] = jnp.sum(selmask * logits, axis=0,
                                         keepdims=True)


def _combine(clusters, tc_sums, sc_part, W, b2):
    return pl.pallas_call(
        _combine_body,
        grid=(1,),
        in_specs=[
            pl.BlockSpec((_B, _N), lambda i: (0, 0)),
            pl.BlockSpec((_B, _C, _D), lambda i: (0, 0, 0)),
            pl.BlockSpec((_WPB, _B, _C, _D), lambda i: (0, 0, 0, 0)),
            pl.BlockSpec((_D, _NUM_CLASSES), lambda i: (0, 0)),
            pl.BlockSpec((1, _NUM_CLASSES), lambda i: (0, 0)),
        ],
        out_specs=pl.BlockSpec((_B, _NUM_CLASSES), lambda i: (0, 0)),
        out_shape=jax.ShapeDtypeStruct((_B, _NUM_CLASSES), jnp.float32),
    )(clusters, tc_sums, sc_part, W, b2)


@jax.jit
def _run_all(inst_feat, clusters_idcs, W, b):
    cl = clusters_idcs.astype(jnp.int32)
    sc_part = _sc_run(inst_feat, cl)
    tc_sums = _tc_run(inst_feat, cl[:, :_NTC].reshape(_B, 1, _NTC))
    out = _combine(cl, tc_sums, sc_part, W,
                   b.reshape(1, _NUM_CLASSES).astype(jnp.float32))
    return out


def kernel(inst_feat, clusters_idcs, W, b):
    return _run_all(inst_feat, clusters_idcs, W, b)


# speedup vs baseline: 3.1290x; 3.0157x over previous
import jax
import jax.numpy as jnp
from jax.experimental import pallas as pl

_B, _N, _D = 8, 4096, 768
_C = 16
_NUM_CLASSES = 2
_THR = 0.8


def _rdd_body(c_ref, x_ref, w_ref, b_ref, o_ref):
    bidx = pl.program_id(0)
    x = x_ref[0]
    cid = jax.lax.broadcasted_iota(jnp.int32, (_C, _N), 0)
    maskb = (c_ref[0] == cid).astype(jnp.bfloat16)
    x_hi = x.astype(jnp.bfloat16)
    x_lo = (x - x_hi.astype(jnp.float32)).astype(jnp.bfloat16)
    sums = jnp.dot(
        maskb, x_hi, preferred_element_type=jnp.float32
    ) + jnp.dot(maskb, x_lo, preferred_element_type=jnp.float32)
    counts = jnp.sum(
        maskb.astype(jnp.float32), axis=1, keepdims=True
    )
    feats = sums / jnp.maximum(counts, 1.0)
    logits = (
        jnp.dot(feats, w_ref[...], preferred_element_type=jnp.float32)
        + b_ref[0]
    )
    d = logits[:, 1:2] - logits[:, 0:1]
    dmax = jnp.max(d)
    dmin = jnp.min(d)
    use_min = jax.nn.sigmoid(dmax) < _THR
    target = jnp.where(use_min, dmin, dmax)
    idxs = jax.lax.broadcasted_iota(jnp.int32, (_C, 1), 0)
    sel = jnp.min(jnp.where(d == target, idxs, _C))
    selmask = (idxs == sel).astype(jnp.float32)
    out = jnp.sum(selmask * logits, axis=0, keepdims=True)
    o_ref[pl.ds(bidx, 1), :] = out


@jax.jit
def _run(inst_feat, clusters, W, b2):
    return pl.pallas_call(
        _rdd_body,
        grid=(_B,),
        in_specs=[
            pl.BlockSpec((1, 1, _N), lambda i: (i, 0, 0)),
            pl.BlockSpec((1, _N, _D), lambda i: (i, 0, 0)),
            pl.BlockSpec((_D, _NUM_CLASSES), lambda i: (0, 0)),
            pl.BlockSpec((1, _NUM_CLASSES), lambda i: (0, 0)),
        ],
        out_specs=pl.BlockSpec((_B, _NUM_CLASSES), lambda i: (0, 0)),
        out_shape=jax.ShapeDtypeStruct((_B, _NUM_CLASSES), jnp.float32),
    )(clusters, inst_feat, W, b2)


def kernel(inst_feat, clusters_idcs, W, b):
    clusters = clusters_idcs.astype(jnp.int32).reshape(_B, 1, _N)
    b2 = b.reshape(1, _NUM_CLASSES).astype(jnp.float32)
    return _run(inst_feat, clusters, W, b2)
